# Initial kernel scaffold; baseline (speedup 1.0000x reference)
import jax
import jax.numpy as jnp
from jax.experimental import pallas as pl


def kernel(inputs, params):
    raise NotImplementedError("write your pallas kernel here")


# speedup vs baseline: 264.0607x; 264.0607x over previous
import functools

import jax
import jax.numpy as jnp
from jax.experimental import pallas as pl

BN_EPS = 1e-5
C = 192
N = 1024
B = 4
N_BLOCKS = 12
NUM_KNN = [9, 9, 10, 11, 12, 13, 13, 14, 15, 16, 17, 18]
DILATIONS = [1, 1, 1, 1, 2, 2, 2, 2, 3, 3, 3, 3]
_BN_SCALE = 1.0 / (1.0 + BN_EPS) ** 0.5


def _bn(y, g, beta):
    return g * (y / jnp.sqrt(1.0 + BN_EPS)) + beta


def _affine_body(x_ref, w_ref, b_ref, g_ref, beta_ref, o_ref, *, relu, bias):
    y = jnp.dot(x_ref[...], w_ref[...], preferred_element_type=jnp.float32)
    if bias:
        y = y + b_ref[...]
    y = _bn(y, g_ref[...], beta_ref[...])
    if relu:
        y = jnp.maximum(y, 0.0)
    o_ref[...] = y


def _affine(x, w, b, g, beta, relu=False):
    R, Cin = x.shape
    Cout = w.shape[1]
    bias = b is not None
    args = (x, w) + ((b,) if bias else ()) + (g, beta)
    if not bias:
        def body(x_ref, w_ref, g_ref, beta_ref, o_ref):
            _affine_body(x_ref, w_ref, None, g_ref, beta_ref, o_ref,
                         relu=relu, bias=False)
    else:
        body = functools.partial(_affine_body, relu=relu, bias=True)
    return pl.pallas_call(
        body,
        out_shape=jax.ShapeDtypeStruct((R, Cout), jnp.float32),
    )(*args)


def _ffn_body(x_ref, xt_ref, w1_ref, b1_ref, g1_ref, be1_ref, w2_ref, b2_ref,
              g2_ref, be2_ref, o_ref):
    h = jnp.dot(x_ref[...], w1_ref[...], preferred_element_type=jnp.float32)
    h = jnp.maximum(_bn(h + b1_ref[...], g1_ref[...], be1_ref[...]), 0.0)
    y = jax.lax.dot_general(w2_ref[...], h, (((1,), (1,)), ((), ())),
                            preferred_element_type=jnp.float32)
    o_ref[...] = _bn(y + b2_ref[...], g2_ref[...], be2_ref[...]) + xt_ref[...]


def _ffn(x, xt, w1, b1, g1, be1, w2, b2, g2, be2):
    R = x.shape[0]
    return pl.pallas_call(
        _ffn_body,
        out_shape=jax.ShapeDtypeStruct((C, R), jnp.float32),
    )(x, xt, w1, b1, g1, be1, w2, b2, g2, be2)


def _gc_body(z_ref, wg_ref, bg_ref, gg_ref, beg_ref, w2_ref, b2_ref, g2_ref,
             be2_ref, t_ref, o_ref):
    y = jnp.dot(z_ref[...], wg_ref[...], preferred_element_type=jnp.float32)
    y = jnp.maximum(_bn(y + bg_ref[...], gg_ref[...], beg_ref[...]), 0.0)
    z = jnp.dot(y, w2_ref[...], preferred_element_type=jnp.float32)
    o_ref[...] = _bn(z + b2_ref[...], g2_ref[...], be2_ref[...]) + t_ref[...]


def _gc_fc2(z, wg, bg, gg, beg, w2, b2, g2, be2, t):
    R = z.shape[0]
    return pl.pallas_call(
        _gc_body,
        out_shape=jax.ShapeDtypeStruct((R, C), jnp.float32),
    )(z, wg, bg, gg, beg, w2, b2, g2, be2, t)


def _pairwise_distance(xt):
    x_inner = -2.0 * jnp.matmul(xt, jnp.swapaxes(xt, 2, 1))
    x_sq = jnp.sum(xt * xt, axis=-1, keepdims=True)
    return x_sq + x_inner + jnp.swapaxes(x_sq, 2, 1)


def _knn_body(xc_ref, o_ref, *, kd, d):
    xr = xc_ref[0]
    norm = jnp.sqrt(jnp.sum(xr * xr, axis=0))
    xc = xr / jnp.maximum(norm, 1e-12)[None, :]
    inner = jax.lax.dot_general(xc, xc, (((0,), (0,)), ((), ())),
                                preferred_element_type=jnp.float32)
    s = jnp.sum(xc * xc, axis=0)
    D = (s[:, None] + (-2.0 * inner)) + s[None, :]
    iota = jax.lax.broadcasted_iota(jnp.int32, (N, N), 1)
    for r in range(kd):
        minv = jnp.min(D, axis=1, keepdims=True)
        cand = jnp.where(D == minv, iota, jnp.int32(N))
        idxm = jnp.min(cand, axis=1, keepdims=True)
        if r % d == 0:
            o_ref[0, :, r // d] = idxm[:, 0]
        D = jnp.where(iota == idxm, jnp.float32(jnp.inf), D)


def _knn_idx(xf, k, d):
    xn = jnp.transpose(xf, (0, 2, 1))
    Cc = xn.shape[1]
    if Cc % 8 != 0:
        pad = 8 - Cc % 8
        xn = jnp.concatenate(
            [xn, jnp.zeros((B, pad, N), jnp.float32)], axis=1)
        Cc += pad
    return pl.pallas_call(
        functools.partial(_knn_body, kd=k * d, d=d),
        grid=(B,),
        in_specs=[pl.BlockSpec((1, Cc, N), lambda b: (b, 0, 0))],
        out_specs=pl.BlockSpec((1, N, k), lambda b: (b, 0, 0)),
        out_shape=jax.ShapeDtypeStruct((B, N, k), jnp.int32),
    )(xn)


def _max_rel(xf, idx):
    Bb, Nn, Cc = xf.shape
    k = idx.shape[-1]
    flat = idx.reshape(Bb, Nn * k)
    g = jnp.take_along_axis(xf, flat[..., None], axis=1).reshape(Bb, Nn, k, Cc)
    return jnp.max(g, axis=2) - xf


def kernel(inputs, params):
    x0 = inputs

    idx = _knn_idx(x0, 9, 1)
    m0 = _max_rel(x0, idx)
    z = jnp.stack([x0, m0], axis=-1).reshape(B * N, 6)
    x = _affine(z, params['head_w'].T, None,
                params['head_g'][None, :], params['head_beta'][None, :],
                relu=True)

    for i in range(N_BLOCKS):
        blk = params['blocks'][i]
        gp, fp = blk['grapher'], blk['ffn']
        k, d = NUM_KNN[i], DILATIONS[i]

        t = x
        h = _affine(x, gp['fc1_w'].T, gp['fc1_b'][None, :],
                    gp['fc1_g'][None, :], gp['fc1_beta'][None, :])
        hf = h.reshape(B, N, C)
        idx = _knn_idx(hf, k, d)
        m = _max_rel(hf, idx).reshape(B * N, C)
        z = jnp.stack([h, m], axis=-1).reshape(B * N, 2 * C)
        x = _gc_fc2(z, gp['gc_w'].T, gp['gc_b'][None, :], gp['gc_g'][None, :],
                    gp['gc_beta'][None, :], gp['fc2_w'].T, gp['fc2_b'][None, :],
                    gp['fc2_g'][None, :], gp['fc2_beta'][None, :], t)

        x = _ffn(x, x.T, fp['fc1_w'].T, fp['fc1_b'][None, :],
                 fp['fc1_g'][None, :], fp['fc1_beta'][None, :], fp['fc2_w'],
                 fp['fc2_b'][:, None], fp['fc2_g'][:, None],
                 fp['fc2_beta'][:, None]).T

    pooled = jnp.mean(x.reshape(B, N, C), axis=1)

    def pred_body(p_ref, w_ref, b_ref, o_ref):
        o_ref[...] = jnp.dot(p_ref[...], w_ref[...],
                             preferred_element_type=jnp.float32) + b_ref[...]

    out = pl.pallas_call(
        pred_body, out_shape=jax.ShapeDtypeStruct((B, 256), jnp.float32),
    )(pooled, params['pred_w'].T, params['pred_b'][None, :])
    return out


# speedup vs baseline: 894.2814x; 3.3867x over previous
import functools

import jax
import jax.numpy as jnp
from jax import lax
from jax.experimental import pallas as pl
from jax.experimental.pallas import tpu as pltpu
from jax.experimental.pallas import tpu_sc as plsc

BN_EPS = 1e-5
C = 192
N = 1024
B = 4
N_BLOCKS = 12
NUM_KNN = [9, 9, 10, 11, 12, 13, 13, 14, 15, 16, 17, 18]
DILATIONS = [1, 1, 1, 1, 2, 2, 2, 2, 3, 3, 3, 3]
_BN_SCALE = 1.0 / (1.0 + BN_EPS) ** 0.5


def _bn(y, g, beta):
    return g * (y / jnp.sqrt(1.0 + BN_EPS)) + beta


def _affine_body(x_ref, w_ref, b_ref, g_ref, beta_ref, o_ref, *, relu, bias):
    y = jnp.dot(x_ref[...], w_ref[...], preferred_element_type=jnp.float32)
    if bias:
        y = y + b_ref[...]
    y = _bn(y, g_ref[...], beta_ref[...])
    if relu:
        y = jnp.maximum(y, 0.0)
    o_ref[...] = y


def _affine(x, w, b, g, beta, relu=False):
    R, Cin = x.shape
    Cout = w.shape[1]
    bias = b is not None
    args = (x, w) + ((b,) if bias else ()) + (g, beta)
    if not bias:
        def body(x_ref, w_ref, g_ref, beta_ref, o_ref):
            _affine_body(x_ref, w_ref, None, g_ref, beta_ref, o_ref,
                         relu=relu, bias=False)
    else:
        body = functools.partial(_affine_body, relu=relu, bias=True)
    return pl.pallas_call(
        body,
        out_shape=jax.ShapeDtypeStruct((R, Cout), jnp.float32),
    )(*args)


def _ffn_body(x_ref, xt_ref, w1_ref, b1_ref, g1_ref, be1_ref, w2_ref, b2_ref,
              g2_ref, be2_ref, o_ref):
    h = jnp.dot(x_ref[...], w1_ref[...], preferred_element_type=jnp.float32)
    h = jnp.maximum(_bn(h + b1_ref[...], g1_ref[...], be1_ref[...]), 0.0)
    y = jax.lax.dot_general(w2_ref[...], h, (((1,), (1,)), ((), ())),
                            preferred_element_type=jnp.float32)
    o_ref[...] = _bn(y + b2_ref[...], g2_ref[...], be2_ref[...]) + xt_ref[...]


def _ffn(x, xt, w1, b1, g1, be1, w2, b2, g2, be2):
    R = x.shape[0]
    return pl.pallas_call(
        _ffn_body,
        out_shape=jax.ShapeDtypeStruct((C, R), jnp.float32),
    )(x, xt, w1, b1, g1, be1, w2, b2, g2, be2)


def _gc_body(z_ref, wg_ref, bg_ref, gg_ref, beg_ref, w2_ref, b2_ref, g2_ref,
             be2_ref, t_ref, o_ref):
    y = jnp.dot(z_ref[...], wg_ref[...], preferred_element_type=jnp.float32)
    y = jnp.maximum(_bn(y + bg_ref[...], gg_ref[...], beg_ref[...]), 0.0)
    z = jnp.dot(y, w2_ref[...], preferred_element_type=jnp.float32)
    o_ref[...] = _bn(z + b2_ref[...], g2_ref[...], be2_ref[...]) + t_ref[...]


def _gc_fc2(z, wg, bg, gg, beg, w2, b2, g2, be2, t):
    R = z.shape[0]
    return pl.pallas_call(
        _gc_body,
        out_shape=jax.ShapeDtypeStruct((R, C), jnp.float32),
    )(z, wg, bg, gg, beg, w2, b2, g2, be2, t)


def _pairwise_distance(xt):
    x_inner = -2.0 * jnp.matmul(xt, jnp.swapaxes(xt, 2, 1))
    x_sq = jnp.sum(xt * xt, axis=-1, keepdims=True)
    return x_sq + x_inner + jnp.swapaxes(x_sq, 2, 1)


def _knn_body(xc_ref, o_ref, *, kd, d):
    xr = xc_ref[0]
    norm = jnp.sqrt(jnp.sum(xr * xr, axis=0))
    xc = xr / jnp.maximum(norm, 1e-12)[None, :]
    inner = jax.lax.dot_general(xc, xc, (((0,), (0,)), ((), ())),
                                preferred_element_type=jnp.float32)
    s = jnp.sum(xc * xc, axis=0)
    D = (s[:, None] + (-2.0 * inner)) + s[None, :]
    iota = jax.lax.broadcasted_iota(jnp.int32, (N, N), 1)
    for r in range(kd):
        minv = jnp.min(D, axis=1, keepdims=True)
        cand = jnp.where(D == minv, iota, jnp.int32(N))
        idxm = jnp.min(cand, axis=1, keepdims=True)
        if r % d == 0:
            o_ref[0, :, r // d] = idxm[:, 0]
        D = jnp.where(iota == idxm, jnp.float32(jnp.inf), D)


def _knn_idx(xf, k, d):
    xn = jnp.transpose(xf, (0, 2, 1))
    Cc = xn.shape[1]
    if Cc % 8 != 0:
        pad = 8 - Cc % 8
        xn = jnp.concatenate(
            [xn, jnp.zeros((B, pad, N), jnp.float32)], axis=1)
        Cc += pad
    return pl.pallas_call(
        functools.partial(_knn_body, kd=k * d, d=d),
        grid=(B,),
        in_specs=[pl.BlockSpec((1, Cc, N), lambda b: (b, 0, 0))],
        out_specs=pl.BlockSpec((1, N, k), lambda b: (b, 0, 0)),
        out_shape=jax.ShapeDtypeStruct((B, N, k), jnp.int32),
    )(xn)


_NW = 32
_RPW = (B * N) // _NW


def _sc_gather_max(h, idxT, k, Cp):
    R = h.shape[0]
    mesh = plsc.VectorSubcoreMesh(core_axis_name="c", subcore_axis_name="s")

    @functools.partial(
        pl.kernel, mesh=mesh,
        out_type=jax.ShapeDtypeStruct((R, Cp), jnp.float32),
        scratch_types=[
            pltpu.VMEM((_RPW,), jnp.int32),
            pltpu.VMEM((_RPW, Cp), jnp.float32),
            pltpu.VMEM((_RPW, Cp), jnp.float32),
            pltpu.VMEM((_RPW, Cp), jnp.float32),
            pltpu.SemaphoreType.DMA,
        ],
    )
    def kern(h_hbm, idxT_hbm, out_hbm, idx_v, buf_v, acc_v, x_v, sem):
        wid = lax.axis_index("s") * 2 + lax.axis_index("c")
        base = wid * _RPW
        pltpu.sync_copy(h_hbm.at[pl.ds(base, _RPW)], x_v)
        pltpu.sync_copy(idxT_hbm.at[pl.ds(base, _RPW)], idx_v)
        pltpu.async_copy(h_hbm.at[idx_v], acc_v, sem).wait()
        nlane = Cp // 16

        for j in range(1, k):
            pltpu.sync_copy(idxT_hbm.at[pl.ds(j * R + base, _RPW)], idx_v)
            pltpu.async_copy(h_hbm.at[idx_v], buf_v, sem).wait()

            def body(r, _):
                for cc in range(nlane):
                    sl = pl.ds(cc * 16, 16)
                    acc_v[r, sl] = jnp.maximum(acc_v[r, sl], buf_v[r, sl])
                return 0

            lax.fori_loop(0, _RPW, body, 0)

        def body2(r, _):
            for cc in range(nlane):
                sl = pl.ds(cc * 16, 16)
                acc_v[r, sl] = acc_v[r, sl] - x_v[r, sl]
            return 0

        lax.fori_loop(0, _RPW, body2, 0)
        pltpu.sync_copy(acc_v, out_hbm.at[pl.ds(base, _RPW)])

    return kern(h, idxT)


def _max_rel(xf, idx):
    Bb, Nn, Cc = xf.shape
    k = idx.shape[-1]
    h = xf.reshape(Bb * Nn, Cc)
    Cp = Cc if Cc % 128 == 0 else Cc + (128 - Cc % 128)
    if Cp != Cc:
        h = jnp.concatenate(
            [h, jnp.zeros((Bb * Nn, Cp - Cc), jnp.float32)], axis=1)
    offs = (jnp.arange(Bb, dtype=jnp.int32) * Nn)[:, None, None]
    idxT = (idx + offs).reshape(Bb * Nn, k).T.reshape(-1)
    m = _sc_gather_max(h, idxT, k, Cp)
    return m[:, :Cc].reshape(Bb, Nn, Cc)


def kernel(inputs, params):
    x0 = inputs

    idx = _knn_idx(x0, 9, 1)
    m0 = _max_rel(x0, idx)
    z = jnp.stack([x0, m0], axis=-1).reshape(B * N, 6)
    x = _affine(z, params['head_w'].T, None,
                params['head_g'][None, :], params['head_beta'][None, :],
                relu=True)

    for i in range(N_BLOCKS):
        blk = params['blocks'][i]
        gp, fp = blk['grapher'], blk['ffn']
        k, d = NUM_KNN[i], DILATIONS[i]

        t = x
        h = _affine(x, gp['fc1_w'].T, gp['fc1_b'][None, :],
                    gp['fc1_g'][None, :], gp['fc1_beta'][None, :])
        hf = h.reshape(B, N, C)
        idx = _knn_idx(hf, k, d)
        m = _max_rel(hf, idx).reshape(B * N, C)
        z = jnp.stack([h, m], axis=-1).reshape(B * N, 2 * C)
        x = _gc_fc2(z, gp['gc_w'].T, gp['gc_b'][None, :], gp['gc_g'][None, :],
                    gp['gc_beta'][None, :], gp['fc2_w'].T, gp['fc2_b'][None, :],
                    gp['fc2_g'][None, :], gp['fc2_beta'][None, :], t)

        x = _ffn(x, x.T, fp['fc1_w'].T, fp['fc1_b'][None, :],
                 fp['fc1_g'][None, :], fp['fc1_beta'][None, :], fp['fc2_w'],
                 fp['fc2_b'][:, None], fp['fc2_g'][:, None],
                 fp['fc2_beta'][:, None]).T

    pooled = jnp.mean(x.reshape(B, N, C), axis=1)

    def pred_body(p_ref, w_ref, b_ref, o_ref):
        o_ref[...] = jnp.dot(p_ref[...], w_ref[...],
                             preferred_element_type=jnp.float32) + b_ref[...]

    out = pl.pallas_call(
        pred_body, out_shape=jax.ShapeDtypeStruct((B, 256), jnp.float32),
    )(pooled, params['pred_w'].T, params['pred_b'][None, :])
    return out
